# baseline (speedup 1.0000x reference)
import functools

import jax
import jax.numpy as jnp
from jax import lax
from jax.experimental import pallas as pl
from jax.experimental.pallas import tpu as pltpu
from jax.experimental.pallas import tpu_sc as plsc

N_FIELDS = 26
VOCAB = 100000
DIM = 32
PAD_DIM = 128
BATCH = 16384

_info = plsc.get_sparse_core_info()
_NC = _info.num_cores
_NS = _info.num_subcores
_NW = _NC * _NS

B_TOTAL = BATCH * N_FIELDS
B_PER_W = B_TOTAL // _NW
CHUNK = 832
N_CHUNKS = B_PER_W // CHUNK

_mesh = plsc.VectorSubcoreMesh(core_axis_name="c", subcore_axis_name="s")


@functools.partial(
    pl.kernel,
    mesh=_mesh,
    out_type=jax.ShapeDtypeStruct((B_TOTAL, DIM), jnp.float32),
    scratch_types=[
        pltpu.VMEM((CHUNK,), jnp.int32),
        pltpu.VMEM((CHUNK, PAD_DIM), jnp.float32),
        pltpu.SemaphoreType.DMA,
    ],
    compiler_params=pltpu.CompilerParams(use_tc_tiling_on_sc=False),
)
def _sc_gather(idx_hbm, table_hbm, out_hbm, idx_v, rows_v, sem):
    wid = lax.axis_index("s") * _NC + lax.axis_index("c")
    base = wid * B_PER_W

    def body(i, _):
        off = pl.multiple_of(base + i * CHUNK, 8)
        pltpu.sync_copy(idx_hbm.at[pl.ds(off, CHUNK)], idx_v)
        pltpu.async_copy(table_hbm.at[idx_v], rows_v, sem).wait()
        pltpu.sync_copy(rows_v.at[:, pl.ds(0, DIM)], out_hbm.at[pl.ds(off, CHUNK)])
        return ()

    lax.fori_loop(0, N_CHUNKS, body, ())


def kernel(value, tables):
    offs = (jnp.arange(N_FIELDS, dtype=jnp.int32) * VOCAB)[None, :]
    idx = (value.astype(jnp.int32) + offs).reshape(B_TOTAL)
    padded = jnp.pad(tables, ((0, 0), (0, 0), (0, PAD_DIM - DIM)))
    flat_tables = padded.reshape(N_FIELDS * VOCAB, PAD_DIM)
    out = _sc_gather(idx, flat_tables)
    return out.reshape(BATCH, N_FIELDS, DIM)


# speedup vs baseline: 1.5884x; 1.5884x over previous
import functools

import jax
import jax.numpy as jnp
from jax import lax
from jax.experimental import pallas as pl
from jax.experimental.pallas import tpu as pltpu
from jax.experimental.pallas import tpu_sc as plsc

N_FIELDS = 26
VOCAB = 100000
DIM = 32
BATCH = 16384

_info = plsc.get_sparse_core_info()
_NC = _info.num_cores
_NS = _info.num_subcores
_NW = _NC * _NS

_WB = 4096
_L = 16

_mesh = plsc.VectorSubcoreMesh(core_axis_name="c", subcore_axis_name="s")


@functools.partial(
    pl.kernel,
    mesh=_mesh,
    out_type=jax.ShapeDtypeStruct((N_FIELDS, DIM, BATCH), jnp.float32),
    scratch_types=[
        pltpu.VMEM((BATCH,), jnp.int32),
        pltpu.VMEM((VOCAB,), jnp.float32),
        pltpu.VMEM((_WB,), jnp.float32),
    ],
    compiler_params=pltpu.CompilerParams(
        use_tc_tiling_on_sc=False, needs_layout_passes=False
    ),
)
def _sc_gather(valt_hbm, tabt_hbm, out_hbm, idx_v, row_v, g_v):
    d = lax.axis_index("s") * _NC + lax.axis_index("c")

    def field_body(f, _):
        pltpu.sync_copy(valt_hbm.at[f], idx_v)
        pltpu.sync_copy(tabt_hbm.at[f, d], row_v)

        def piece_body(c, _):
            def vec_body(i, _):
                iv = idx_v[pl.ds(c * _WB + i * _L, _L)]
                g_v[pl.ds(i * _L, _L)] = plsc.load_gather(row_v, [iv])
                return ()

            lax.fori_loop(0, _WB // _L, vec_body, (), unroll=4)
            pltpu.sync_copy(g_v, out_hbm.at[f, d, pl.ds(c * _WB, _WB)])
            return ()

        lax.fori_loop(0, BATCH // _WB, piece_body, ())
        return ()

    lax.fori_loop(0, N_FIELDS, field_body, ())


def kernel(value, tables):
    valt = value.astype(jnp.int32).T
    tabt = jnp.transpose(tables, (0, 2, 1))
    out = _sc_gather(valt, tabt)
    return jnp.transpose(out, (2, 0, 1))


# speedup vs baseline: 3.4807x; 2.1914x over previous
import functools

import jax
import jax.numpy as jnp
from jax import lax
from jax.experimental import pallas as pl
from jax.experimental.pallas import tpu as pltpu
from jax.experimental.pallas import tpu_sc as plsc

N_FIELDS = 26
VOCAB = 100000
DIM = 32
BATCH = 16384

_info = plsc.get_sparse_core_info()
_NC = _info.num_cores
_NS = _info.num_subcores
_NW = _NC * _NS

_WB = 4096
_L = 16

_mesh = plsc.VectorSubcoreMesh(core_axis_name="c", subcore_axis_name="s")


@functools.partial(
    pl.kernel,
    mesh=_mesh,
    out_type=jax.ShapeDtypeStruct((N_FIELDS, DIM, BATCH), jnp.float32),
    scratch_types=[
        pltpu.VMEM((BATCH,), jnp.int32),
        pltpu.VMEM((VOCAB,), jnp.float32),
        pltpu.VMEM((_WB,), jnp.float32),
    ],
    compiler_params=pltpu.CompilerParams(
        use_tc_tiling_on_sc=True, needs_layout_passes=False
    ),
)
def _sc_gather(valt_hbm, tabt_hbm, out_hbm, idx_v, row_v, g_v):
    d = lax.axis_index("s") * _NC + lax.axis_index("c")

    def field_body(f, _):
        pltpu.sync_copy(valt_hbm.at[f], idx_v)
        pltpu.sync_copy(tabt_hbm.at[f, d], row_v)

        def piece_body(c, _):
            def vec_body(i, _):
                iv = idx_v[pl.ds(c * _WB + i * _L, _L)]
                g_v[pl.ds(i * _L, _L)] = plsc.load_gather(row_v, [iv])
                return ()

            lax.fori_loop(0, _WB // _L, vec_body, (), unroll=8)
            pltpu.sync_copy(g_v, out_hbm.at[f, d, pl.ds(c * _WB, _WB)])
            return ()

        lax.fori_loop(0, BATCH // _WB, piece_body, ())
        return ()

    lax.fori_loop(0, N_FIELDS, field_body, ())


def kernel(value, tables):
    valt = value.astype(jnp.int32).T
    tabt = jnp.transpose(tables, (0, 2, 1))
    out = _sc_gather(valt, tabt)
    return jnp.transpose(out, (2, 0, 1))


# speedup vs baseline: 7.0990x; 2.0395x over previous
import functools

import jax
import jax.numpy as jnp
from jax import lax
from jax.experimental import pallas as pl
from jax.experimental.pallas import tpu as pltpu
from jax.experimental.pallas import tpu_sc as plsc

N_FIELDS = 26
VOCAB = 100000
DIM = 32
BATCH = 16384

_info = plsc.get_sparse_core_info()
_NC = _info.num_cores
_NS = _info.num_subcores
_NW = _NC * _NS

_WB = 8192
_L = 16

_mesh = plsc.VectorSubcoreMesh(core_axis_name="c", subcore_axis_name="s")


@functools.partial(
    pl.kernel,
    mesh=_mesh,
    out_type=jax.ShapeDtypeStruct((N_FIELDS, DIM, BATCH), jnp.float32),
    scratch_types=[
        pltpu.VMEM((BATCH,), jnp.int32),
        pltpu.VMEM((VOCAB,), jnp.float32),
        pltpu.VMEM((_WB,), jnp.float32),
    ],
    compiler_params=pltpu.CompilerParams(
        use_tc_tiling_on_sc=True, needs_layout_passes=False
    ),
)
def _sc_gather(valt_hbm, tabt_hbm, out_hbm, idx_v, row_v, g_v):
    d = lax.axis_index("s") * _NC + lax.axis_index("c")

    def field_body(f, _):
        pltpu.sync_copy(valt_hbm.at[f], idx_v)
        pltpu.sync_copy(tabt_hbm.at[f, d], row_v)

        def piece_body(c, _):
            @functools.partial(plsc.parallel_loop, 0, _WB // _L, unroll=8)
            def vec_body(i):
                iv = idx_v[pl.ds(c * _WB + i * _L, _L)]
                g_v[pl.ds(i * _L, _L)] = plsc.load_gather(row_v, [iv])

            pltpu.sync_copy(g_v, out_hbm.at[f, d, pl.ds(c * _WB, _WB)])
            return ()

        lax.fori_loop(0, BATCH // _WB, piece_body, ())
        return ()

    lax.fori_loop(0, N_FIELDS, field_body, ())


def kernel(value, tables):
    valt = value.astype(jnp.int32).T
    tabt = jnp.transpose(tables, (0, 2, 1))
    out = _sc_gather(valt, tabt)
    return jnp.transpose(out, (2, 0, 1))
